# baseline (speedup 1.0000x reference)
import jax
import jax.numpy as jnp
from jax import lax
from jax.experimental import pallas as pl
from jax.experimental.pallas import tpu as pltpu
from jax.experimental.pallas import tpu_sc as plsc

N_ROWS = 1000000
K = 64
BATCH = 16384

NC = 2
NS = 16
NW = NC * NS
B_PER_W = BATCH // NW
BLOCKS_PER_W = 256
NGRP = 16
GBLK = BLOCKS_PER_W // NGRP
NIDV = BATCH // 16
LST_CAP = 736
SUB_CAP = 96
MB_CAP = 64
RING = 8


def _collect(ids_v, lo, hi, lanes, lst_id, lst_pos, bcnt, toff):
    ones = jnp.full((16,), 1, jnp.int32)

    def coll(i, cnt):
        v = ids_v[pl.ds(i * 16, 16)]
        m = (v >= lo) & (v < hi)
        plsc.store_compressed(lst_id.at[pl.ds(cnt, 16)], v, mask=m)
        plsc.store_compressed(lst_pos.at[pl.ds(cnt, 16)],
                              i * 16 + lanes, mask=m)
        plsc.addupdate_scatter(bcnt, [toff + (v - lo) // 128], ones, mask=m)
        return cnt + plsc.all_reduce_population_count(m)[0]
    return lax.fori_loop(0, NIDV, coll, 0)


def _gather_body(uids_hbm, iids_hbm, umat_t, imat_t, ustage, istage,
                 ids_u, ids_i, lst_id, lst_pos, sub_id, sub_pos, gcnt_v,
                 bcnt, ring, mb_l, mb_p, rowflat, sem_blk, sem_out):
    wid = lax.axis_index("s") * NC + lax.axis_index("c")
    lo = wid * (BLOCKS_PER_W * 128)
    hi = jnp.minimum(lo + BLOCKS_PER_W * 128, N_ROWS)
    nb = jnp.maximum((hi - lo + 127) // 128, 0)
    lanes = lax.iota(jnp.int32, 16)

    pltpu.sync_copy(uids_hbm, ids_u)
    pltpu.sync_copy(iids_hbm, ids_i)

    zeros16 = jnp.zeros((16,), jnp.int32)

    def zinit(i, carry):
        bcnt[pl.ds(i * 16, 16)] = zeros16
        return carry
    lax.fori_loop(0, 2 * BLOCKS_PER_W // 16, zinit, 0)

    def blk_src(mat_t, c):
        off = pl.multiple_of((wid * BLOCKS_PER_W + c) * 128, 128)
        return mat_t.at[:, pl.ds(off, 128)]

    for b in range(RING):
        @pl.when(b < nb)
        def _(b=b):
            pltpu.async_copy(blk_src(umat_t, b), ring.at[b], sem_blk)

    for t, ids_v in ((0, ids_u), (1, ids_i)):
        cnt = _collect(ids_v, lo, hi, lanes, lst_id, lst_pos,
                       bcnt, t * BLOCKS_PER_W)
        nv = (cnt + 15) // 16
        tbase = t * NGRP * SUB_CAP

        def bucket(grp, carry, tbase=tbase, cnt=cnt, nv=nv, t=t):
            glo = lo + grp * (GBLK * 128)
            ghi = glo + GBLK * 128
            gbase = tbase + grp * SUB_CAP

            def buck(j, gc):
                idv = lst_id[pl.ds(j * 16, 16)]
                posv = lst_pos[pl.ds(j * 16, 16)]
                m = ((j * 16 + lanes) < cnt) & (idv >= glo) & (idv < ghi)
                plsc.store_compressed(sub_id.at[pl.ds(gbase + gc, 16)],
                                      idv, mask=m)
                plsc.store_compressed(sub_pos.at[pl.ds(gbase + gc, 16)],
                                      posv, mask=m)
                return gc + plsc.all_reduce_population_count(m)[0]
            gc = lax.fori_loop(0, nv, buck, 0)
            plsc.store_scatter(gcnt_v,
                               [jnp.full((16,), t * NGRP + grp, jnp.int32)],
                               jnp.full((16,), gc, jnp.int32),
                               mask=lanes == 0)
            return carry
        lax.fori_loop(0, NGRP, bucket, 0)

    for t, mat_t, stage in ((0, umat_t, ustage), (1, imat_t, istage)):
        if t == 1:
            for b in range(RING):
                @pl.when(b < nb)
                def _(b=b):
                    pltpu.async_copy(blk_src(imat_t, b), ring.at[b],
                                     sem_blk)
        tbase = t * NGRP * SUB_CAP

        def process(c, buf, tbase=tbase, t=t):
            c_lo = lo + c * 128
            grp = c // GBLK
            gc = plsc.load_gather(
                gcnt_v, [jnp.full((16,), t * NGRP + grp, jnp.int32)])[0]
            gbase = tbase + grp * SUB_CAP

            def scanj(j, nm):
                idv = sub_id[pl.ds(gbase + j * 16, 16)]
                posv = sub_pos[pl.ds(gbase + j * 16, 16)]
                m = ((j * 16 + lanes) < gc) & (idv >= c_lo) & \
                    (idv < c_lo + 128)
                plsc.store_compressed(mb_l.at[pl.ds(nm, 16)],
                                      idv - c_lo, mask=m)
                plsc.store_compressed(mb_p.at[pl.ds(nm, 16)], posv, mask=m)
                return nm + plsc.all_reduce_population_count(m)[0]
            nm = lax.fori_loop(0, (gc + 15) // 16, scanj, 0)

            def match(m, carry):
                msplat = jnp.full((16,), m, jnp.int32)
                l_s = plsc.load_gather(mb_l, [msplat])[0]
                p_s = plsc.load_gather(mb_p, [msplat])[0]
                slot64 = (m % 32) * K
                lsplat = jnp.full((16,), l_s, jnp.int32)
                for j4 in range(4):
                    rv = plsc.load_gather(buf, [j4 * 16 + lanes, lsplat])
                    rowflat[pl.ds(slot64 + j4 * 16, 16)] = rv
                pltpu.async_copy(
                    rowflat.at[pl.ds(slot64, K)],
                    stage.at[pl.ds(p_s * K, K)], sem_out)
                return carry
            lax.fori_loop(0, nm, match, 0)

            def drain(d, carry):
                pltpu.make_async_copy(
                    rowflat.at[pl.ds(0, K)],
                    stage.at[pl.ds(0, K)], sem_out).wait()
                return carry
            lax.fori_loop(0, nm, drain, 0)

        boff = t * BLOCKS_PER_W

        def hits(c):
            return plsc.load_gather(
                bcnt, [jnp.full((16,), boff + c, jnp.int32)])[0] > 0

        def ginner(gg, carry, mat_t=mat_t, process=process, boff=boff,
                   hits=hits):
            for b in range(RING):
                c = gg * RING + b

                @pl.when((c < nb) & ((c < RING) | hits(c)))
                def _(b=b, c=c):
                    pltpu.make_async_copy(blk_src(mat_t, c), ring.at[b],
                                          sem_blk).wait()
                    process(c, ring.at[b])

                @pl.when((c + RING < nb) & hits(c + RING))
                def _(b=b, c=c):
                    pltpu.async_copy(blk_src(mat_t, c + RING),
                                     ring.at[b], sem_blk)
            return carry
        lax.fori_loop(0, BLOCKS_PER_W // RING, ginner, 0)


def _dot_body(ustage, istage, out_hbm, u_v, i_v, out_v, sem):
    wid = lax.axis_index("s") * NC + lax.axis_index("c")
    base = wid * B_PER_W
    cu = pltpu.async_copy(ustage.at[pl.ds(base * K, B_PER_W * K)], u_v, sem)
    ci = pltpu.async_copy(istage.at[pl.ds(base * K, B_PER_W * K)], i_v, sem)
    cu.wait()
    ci.wait()
    lanes = lax.iota(jnp.int32, 16)

    def group(g, carry):
        row0 = pl.multiple_of(g * 16, 16)
        flat0 = (row0 + lanes) * K
        acc = [jnp.zeros((16,), jnp.float32) for _ in range(4)]
        for k in range(K):
            u = plsc.load_gather(u_v, [flat0 + k])
            v = plsc.load_gather(i_v, [flat0 + k])
            acc[k % 4] = acc[k % 4] + u * v
        out_v[pl.ds(row0, 16)] = (acc[0] + acc[1]) + (acc[2] + acc[3])
        return carry
    lax.fori_loop(0, B_PER_W // 16, group, 0)

    pltpu.sync_copy(out_v, out_hbm.at[pl.ds(base, B_PER_W)])


def kernel(user_ids, item_ids, user_matrix, item_matrix):
    mesh = plsc.VectorSubcoreMesh(core_axis_name="c", subcore_axis_name="s")
    params = pltpu.CompilerParams(
        needs_layout_passes=False, use_tc_tiling_on_sc=True)

    gather = pl.kernel(
        _gather_body,
        mesh=mesh,
        out_type=(jax.ShapeDtypeStruct((BATCH * K,), jnp.float32),
                  jax.ShapeDtypeStruct((BATCH * K,), jnp.float32)),
        scratch_types=[
            pltpu.VMEM((BATCH,), jnp.int32),
            pltpu.VMEM((BATCH,), jnp.int32),
            pltpu.VMEM((LST_CAP,), jnp.int32),
            pltpu.VMEM((LST_CAP,), jnp.int32),
            pltpu.VMEM((2 * NGRP * SUB_CAP,), jnp.int32),
            pltpu.VMEM((2 * NGRP * SUB_CAP,), jnp.int32),
            pltpu.VMEM((2 * NGRP,), jnp.int32),
            pltpu.VMEM((2 * BLOCKS_PER_W,), jnp.int32),
            pltpu.VMEM((RING, K, 128), jnp.float32),
            pltpu.VMEM((MB_CAP,), jnp.int32),
            pltpu.VMEM((MB_CAP,), jnp.int32),
            pltpu.VMEM((32 * K,), jnp.float32),
            pltpu.SemaphoreType.DMA,
            pltpu.SemaphoreType.DMA,
        ],
        compiler_params=params,
    )

    dot = pl.kernel(
        _dot_body,
        mesh=mesh,
        out_type=jax.ShapeDtypeStruct((BATCH,), jnp.float32),
        scratch_types=[
            pltpu.VMEM((B_PER_W * K,), jnp.float32),
            pltpu.VMEM((B_PER_W * K,), jnp.float32),
            pltpu.VMEM((B_PER_W,), jnp.float32),
            pltpu.SemaphoreType.DMA,
        ],
        compiler_params=params,
    )

    ustage, istage = gather(user_ids.astype(jnp.int32),
                            item_ids.astype(jnp.int32),
                            user_matrix.T, item_matrix.T)
    return dot(ustage, istage)


# speedup vs baseline: 1.2220x; 1.2220x over previous
import jax
import jax.numpy as jnp
from jax import lax
from jax.experimental import pallas as pl
from jax.experimental.pallas import tpu as pltpu
from jax.experimental.pallas import tpu_sc as plsc

N_ROWS = 1000000
K = 64
BATCH = 16384

NC = 2
NS = 16
NW = NC * NS
B_PER_W = BATCH // NW
BLOCKS_PER_W = 256
NGRP = 16
GBLK = BLOCKS_PER_W // NGRP
NIDV = BATCH // 16
LST_CAP = 736
SUB_CAP = 96
MB_CAP = 64
RING = 8


def _collect(ids_v, lo, hi, lanes, lst_id, lst_pos):
    def coll(i, cnt):
        v = ids_v[pl.ds(i * 16, 16)]
        m = (v >= lo) & (v < hi)
        plsc.store_compressed(lst_id.at[pl.ds(cnt, 16)], v, mask=m)
        plsc.store_compressed(lst_pos.at[pl.ds(cnt, 16)],
                              i * 16 + lanes, mask=m)
        return cnt + plsc.all_reduce_population_count(m)[0]
    return lax.fori_loop(0, NIDV, coll, 0)


def _gather_body(uids_hbm, iids_hbm, umat_t, imat_t, ustage, istage,
                 ids_u, ids_i, lst_id, lst_pos, sub_id, sub_pos, gcnt_v,
                 ring, mb_l, mb_p, rowflat, sem_blk, sem_out):
    wid = lax.axis_index("s") * NC + lax.axis_index("c")
    lo = wid * (BLOCKS_PER_W * 128)
    hi = jnp.minimum(lo + BLOCKS_PER_W * 128, N_ROWS)
    nb = jnp.maximum((hi - lo + 127) // 128, 0)
    lanes = lax.iota(jnp.int32, 16)

    pltpu.sync_copy(uids_hbm, ids_u)
    pltpu.sync_copy(iids_hbm, ids_i)

    def blk_src(mat_t, c):
        off = pl.multiple_of((wid * BLOCKS_PER_W + c) * 128, 128)
        return mat_t.at[:, pl.ds(off, 128)]

    for b in range(RING):
        @pl.when(b < nb)
        def _(b=b):
            pltpu.async_copy(blk_src(umat_t, b), ring.at[b], sem_blk)

    for t, ids_v in ((0, ids_u), (1, ids_i)):
        cnt = _collect(ids_v, lo, hi, lanes, lst_id, lst_pos)
        nv = (cnt + 15) // 16
        tbase = t * NGRP * SUB_CAP

        def bucket(grp, carry, tbase=tbase, cnt=cnt, nv=nv, t=t):
            glo = lo + grp * (GBLK * 128)
            ghi = glo + GBLK * 128
            gbase = tbase + grp * SUB_CAP

            def buck(j, gc):
                idv = lst_id[pl.ds(j * 16, 16)]
                posv = lst_pos[pl.ds(j * 16, 16)]
                m = ((j * 16 + lanes) < cnt) & (idv >= glo) & (idv < ghi)
                plsc.store_compressed(sub_id.at[pl.ds(gbase + gc, 16)],
                                      idv, mask=m)
                plsc.store_compressed(sub_pos.at[pl.ds(gbase + gc, 16)],
                                      posv, mask=m)
                return gc + plsc.all_reduce_population_count(m)[0]
            gc = lax.fori_loop(0, nv, buck, 0)
            plsc.store_scatter(gcnt_v,
                               [jnp.full((16,), t * NGRP + grp, jnp.int32)],
                               jnp.full((16,), gc, jnp.int32),
                               mask=lanes == 0)
            return carry
        lax.fori_loop(0, NGRP, bucket, 0)

    for t, mat_t, stage in ((0, umat_t, ustage), (1, imat_t, istage)):
        if t == 1:
            for b in range(RING):
                @pl.when(b < nb)
                def _(b=b):
                    pltpu.async_copy(blk_src(imat_t, b), ring.at[b],
                                     sem_blk)
        tbase = t * NGRP * SUB_CAP

        def process(c, buf, tbase=tbase, t=t):
            c_lo = lo + c * 128
            grp = c // GBLK
            gc = plsc.load_gather(
                gcnt_v, [jnp.full((16,), t * NGRP + grp, jnp.int32)])[0]
            gbase = tbase + grp * SUB_CAP

            def scanj(j, nm):
                idv = sub_id[pl.ds(gbase + j * 16, 16)]
                posv = sub_pos[pl.ds(gbase + j * 16, 16)]
                m = ((j * 16 + lanes) < gc) & (idv >= c_lo) & \
                    (idv < c_lo + 128)
                plsc.store_compressed(mb_l.at[pl.ds(nm, 16)],
                                      idv - c_lo, mask=m)
                plsc.store_compressed(mb_p.at[pl.ds(nm, 16)], posv, mask=m)
                return nm + plsc.all_reduce_population_count(m)[0]
            nm = lax.fori_loop(0, (gc + 15) // 16, scanj, 0)

            def match(m, carry):
                msplat = jnp.full((16,), m, jnp.int32)
                l_s = plsc.load_gather(mb_l, [msplat])[0]
                p_s = plsc.load_gather(mb_p, [msplat])[0]
                slot64 = (m % 32) * K
                lsplat = jnp.full((16,), l_s, jnp.int32)
                for j4 in range(4):
                    rv = plsc.load_gather(buf, [j4 * 16 + lanes, lsplat])
                    rowflat[pl.ds(slot64 + j4 * 16, 16)] = rv
                pltpu.async_copy(
                    rowflat.at[pl.ds(slot64, K)],
                    stage.at[pl.ds(p_s * 128, K)], sem_out)
                return carry
            lax.fori_loop(0, nm, match, 0)

            def drain(d, carry):
                pltpu.make_async_copy(
                    rowflat.at[pl.ds(0, K)],
                    stage.at[pl.ds(0, K)], sem_out).wait()
                return carry
            lax.fori_loop(0, nm, drain, 0)

        def ginner(gg, carry, mat_t=mat_t, process=process):
            for b in range(RING):
                c = gg * RING + b

                @pl.when(c < nb)
                def _(b=b, c=c):
                    pltpu.make_async_copy(blk_src(mat_t, c), ring.at[b],
                                          sem_blk).wait()
                    process(c, ring.at[b])

                    @pl.when(c + RING < nb)
                    def _(b=b, c=c):
                        pltpu.async_copy(blk_src(mat_t, c + RING),
                                         ring.at[b], sem_blk)
            return carry
        lax.fori_loop(0, BLOCKS_PER_W // RING, ginner, 0)


def _tc_dot_body(u_ref, v_ref, o_ref):
    u = u_ref[:, :K]
    v = v_ref[:, :K]
    o_ref[:] = (u * v).sum(axis=1)


def kernel(user_ids, item_ids, user_matrix, item_matrix):
    mesh = plsc.VectorSubcoreMesh(core_axis_name="c", subcore_axis_name="s")
    params = pltpu.CompilerParams(
        needs_layout_passes=False, use_tc_tiling_on_sc=True)

    gather = pl.kernel(
        _gather_body,
        mesh=mesh,
        out_type=(jax.ShapeDtypeStruct((BATCH * 128,), jnp.float32),
                  jax.ShapeDtypeStruct((BATCH * 128,), jnp.float32)),
        scratch_types=[
            pltpu.VMEM((BATCH,), jnp.int32),
            pltpu.VMEM((BATCH,), jnp.int32),
            pltpu.VMEM((LST_CAP,), jnp.int32),
            pltpu.VMEM((LST_CAP,), jnp.int32),
            pltpu.VMEM((2 * NGRP * SUB_CAP,), jnp.int32),
            pltpu.VMEM((2 * NGRP * SUB_CAP,), jnp.int32),
            pltpu.VMEM((2 * NGRP,), jnp.int32),
            pltpu.VMEM((RING, K, 128), jnp.float32),
            pltpu.VMEM((MB_CAP,), jnp.int32),
            pltpu.VMEM((MB_CAP,), jnp.int32),
            pltpu.VMEM((32 * K,), jnp.float32),
            pltpu.SemaphoreType.DMA,
            pltpu.SemaphoreType.DMA,
        ],
        compiler_params=params,
    )

    CH = 1024
    dot = pl.pallas_call(
        _tc_dot_body,
        grid=(BATCH // CH,),
        in_specs=[
            pl.BlockSpec((CH, 128), lambda i: (i, 0)),
            pl.BlockSpec((CH, 128), lambda i: (i, 0)),
        ],
        out_specs=pl.BlockSpec((CH,), lambda i: (i,)),
        out_shape=jax.ShapeDtypeStruct((BATCH,), jnp.float32),
    )

    ustage, istage = gather(user_ids.astype(jnp.int32),
                            item_ids.astype(jnp.int32),
                            user_matrix.T, item_matrix.T)
    return dot(ustage.reshape(BATCH, 128), istage.reshape(BATCH, 128))
